# Initial kernel scaffold; baseline (speedup 1.0000x reference)
import jax
import jax.numpy as jnp
from jax.experimental import pallas as pl


def kernel(lidar_state, position_state, w_slab, b_slab):
    raise NotImplementedError("write your pallas kernel here")


# speedup vs baseline: 3.8627x; 3.8627x over previous
import jax
import jax.numpy as jnp
from jax.experimental import pallas as pl
from jax.experimental.pallas import tpu as pltpu

_LIDAR_DIM = 1080
_POS_DIM = 16
_FEAT_DIM = 64
_HID_DIM = 256
_TB = 1024


def _mlp_kernel(x_ref, pos_ref, w1_ref, b1_ref, w2_ref, b2_ref,
                w3_ref, b3_ref, w4a_ref, w4b_ref, b4_ref,
                w5_ref, b5_ref, out_ref):
    x = x_ref[...]
    h = jnp.dot(x, w1_ref[...], preferred_element_type=jnp.float32) + b1_ref[...]
    h = jnp.maximum(h, 0.0)
    h = jnp.dot(h, w2_ref[...], preferred_element_type=jnp.float32) + b2_ref[...]
    h = jnp.maximum(h, 0.0)
    lf = jnp.dot(h, w3_ref[...], preferred_element_type=jnp.float32) + b3_ref[...]
    t = (jnp.dot(lf, w4a_ref[...], preferred_element_type=jnp.float32)
         + jnp.dot(pos_ref[...], w4b_ref[...], preferred_element_type=jnp.float32)
         + b4_ref[...])
    p = jnp.tanh(t)
    p = jnp.tanh(jnp.dot(p, w5_ref[...], preferred_element_type=jnp.float32)
                 + b5_ref[...])
    out_ref[...] = p


def kernel(lidar_state, position_state, w_slab, b_slab):
    B, L = lidar_state.shape
    H, F, POSD = _HID_DIM, _FEAT_DIM, _POS_DIM
    pos_off = L

    w1 = w_slab[0, :L, :H]
    w2 = w_slab[1, :H, :H]
    w3 = w_slab[2, :H, :F]
    w4a = w_slab[3, :F, :H]
    w4b = w_slab[3, pos_off:pos_off + POSD, :H]
    w5 = w_slab[4, :H, :H]
    b1 = b_slab[0, :, :H]
    b2 = b_slab[1, :, :H]
    b3 = b_slab[2, :, :F]
    b4 = b_slab[3, :, :H]
    b5 = b_slab[4, :, :H]

    TB = min(_TB, B)
    assert B % TB == 0
    const2 = lambda shape: pl.BlockSpec(shape, lambda b: (0, 0))

    out = pl.pallas_call(
        _mlp_kernel,
        out_shape=jax.ShapeDtypeStruct((B, H), jnp.float32),
        grid=(B // TB,),
        in_specs=[
            pl.BlockSpec((TB, L), lambda b: (b, 0)),
            pl.BlockSpec((TB, POSD), lambda b: (b, 0)),
            const2((L, H)), const2((1, H)),
            const2((H, H)), const2((1, H)),
            const2((H, F)), const2((1, F)),
            const2((F, H)), const2((POSD, H)), const2((1, H)),
            const2((H, H)), const2((1, H)),
        ],
        out_specs=pl.BlockSpec((TB, H), lambda b: (b, 0)),
        compiler_params=pltpu.CompilerParams(
            dimension_semantics=("parallel",)),
    )(lidar_state, position_state, w1, b1, w2, b2, w3, b3,
      w4a, w4b, b4, w5, b5)
    return out


# speedup vs baseline: 4.2426x; 1.0983x over previous
import jax
import jax.numpy as jnp
from jax.experimental import pallas as pl
from jax.experimental.pallas import tpu as pltpu

_POS_DIM = 16
_HID_DIM = 256
_TB = 1024


def _mlp_kernel(x_ref, pos_ref, w1_ref, w2_ref, w3_ref, w4a_ref, w4b_ref,
                w5_ref, b1_ref, b2_ref, b3_ref, b4_ref, b5_ref, out_ref):
    x = x_ref[...]
    h = jnp.dot(x, w1_ref[0], preferred_element_type=jnp.float32) + b1_ref[0]
    h = jnp.maximum(h, 0.0)
    h = jnp.dot(h, w2_ref[0], preferred_element_type=jnp.float32) + b2_ref[0]
    h = jnp.maximum(h, 0.0)
    lf = jnp.dot(h, w3_ref[0], preferred_element_type=jnp.float32) + b3_ref[0]
    w4b = w4b_ref[0][:_POS_DIM, :]
    t = (jnp.dot(lf, w4a_ref[0], preferred_element_type=jnp.float32)
         + jnp.dot(pos_ref[...], w4b, preferred_element_type=jnp.float32)
         + b4_ref[0])
    p = jnp.tanh(t)
    p = jnp.tanh(jnp.dot(p, w5_ref[0], preferred_element_type=jnp.float32)
                 + b5_ref[0])
    out_ref[...] = p


def kernel(lidar_state, position_state, w_slab, b_slab):
    B, L = lidar_state.shape
    H, POSD = _HID_DIM, _POS_DIM

    TB = min(_TB, B)
    assert B % TB == 0

    def wspec(layer, rows, cols, row_block=0):
        return pl.BlockSpec((1, rows, cols),
                            lambda b, la=layer, rb=row_block: (la, rb, 0))

    def bspec(layer, cols):
        return pl.BlockSpec((1, 1, cols), lambda b, la=layer: (la, 0, 0))

    out = pl.pallas_call(
        _mlp_kernel,
        out_shape=jax.ShapeDtypeStruct((B, H), jnp.float32),
        grid=(B // TB,),
        in_specs=[
            pl.BlockSpec((TB, L), lambda b: (b, 0)),
            pl.BlockSpec((TB, POSD), lambda b: (b, 0)),
            wspec(0, L, H),
            wspec(1, H, H),
            wspec(2, H, 128),
            wspec(3, 128, H),
            wspec(3, 72, H, 15),
            wspec(4, H, H),
            bspec(0, H), bspec(1, H), bspec(2, 128), bspec(3, H), bspec(4, H),
        ],
        out_specs=pl.BlockSpec((TB, H), lambda b: (b, 0)),
        compiler_params=pltpu.CompilerParams(
            dimension_semantics=("parallel",)),
    )(lidar_state, position_state,
      w_slab, w_slab, w_slab, w_slab, w_slab, w_slab,
      b_slab, b_slab, b_slab, b_slab, b_slab)
    return out
